# baseline (speedup 1.0000x reference)
import functools

import jax
import jax.numpy as jnp
import numpy as np
from jax import lax
from jax.experimental import pallas as pl
from jax.experimental.pallas import tpu as pltpu
from jax.experimental.pallas import tpu_sc as plsc

_VOCAB_SIZES = [100003, 100019, 100043, 100049, 100057, 100069, 100103, 100109]
_OFFSETS = [int(x) for x in np.cumsum([0] + _VOCAB_SIZES[:-1])]

_NUM_CORES = 2
_NUM_SUBCORES = 16
_NUM_WORKERS = _NUM_CORES * _NUM_SUBCORES
_LANES = 16
_H = 8


@functools.partial(jax.jit, static_argnames=("b", "s", "h", "d"))
def _mhe_lookup(ids, table_t8, *, b, s, h, d):
    n = b * s * h
    v = table_t8.shape[0] * 8 // d
    n_per_w = n // _NUM_WORKERS
    s_per_w = n_per_w // h
    rows_per_w = n_per_w // 4
    mesh = plsc.VectorSubcoreMesh(core_axis_name="c", subcore_axis_name="s")

    @functools.partial(
        pl.kernel,
        mesh=mesh,
        out_type=jax.ShapeDtypeStruct((n // 4, 128), jnp.float32),
        scratch_types=[
            pltpu.VMEM((s_per_w, h), jnp.int32),
            pltpu.VMEM((2 * n_per_w,), jnp.int32),
            pltpu.VMEM((2 * n_per_w,), jnp.int32),
            pltpu.VMEM((n_per_w,), jnp.int32),
            pltpu.VMEM((2 * n_per_w, 8), jnp.float32),
            pltpu.VMEM((rows_per_w, 128), jnp.float32),
            pltpu.SemaphoreType.DMA,
        ],
        compiler_params=pltpu.CompilerParams(
            use_tc_tiling_on_sc=False, needs_layout_passes=False
        ),
    )
    def k(ids_hbm, tab_hbm, out_hbm, ids_v, mrow_v, lane_v, idx_v, gbuf_v,
          wbuf_v, sem):
        wid = lax.axis_index("s") * _NUM_CORES + lax.axis_index("c")
        bi = wid // _H
        s0 = (wid % _H) * s_per_w

        pltpu.sync_copy(ids_hbm.at[bi, pl.ds(s0, s_per_w)], ids_v)

        iota = lax.iota(jnp.int32, _LANES)
        for hh in range(h):
            cols_i = jnp.full((_LANES,), hh, jnp.int32)
            for u in range(2):
                row = 2 * hh + u
                for t in range(8):
                    rows_i = iota + (128 * u + 16 * t)
                    f0 = plsc.load_gather(ids_v, [rows_i, cols_i]) + (
                        _OFFSETS[hh]
                    )
                    f1 = f0 + 4
                    sl = pl.ds(row * 128 + 16 * t, _LANES)
                    slo = pl.ds(n_per_w + row * 128 + 16 * t, _LANES)
                    mrow_v[sl] = f0 >> 3
                    mrow_v[slo] = f1 >> 3
                    lane_v[sl] = f0 & 7
                    lane_v[slo] = f1 & 7

        two_iota = 2 * iota
        n_chunks = n_per_w // _LANES

        def extract(dd, par, gbase):
            col_d = jnp.zeros((_LANES,), jnp.int32) + dd
            for hh in range(h):
                ccol = col_d + 32 * (hh % 4)
                for u in range(2):
                    row = 2 * hh + u
                    base = 256 * u + hh // 4
                    for t in range(8):
                        pos = row * 128 + 16 * t
                        grows = gbase + pos + iota
                        lsel = lane_v[pl.ds(par * n_per_w + pos, _LANES)]
                        vals = plsc.load_gather(gbuf_v, [grows, lsel])
                        rws = two_iota + (base + 32 * t)
                        plsc.store_scatter(wbuf_v, [rws, ccol], vals)

        def step(dd, carry):
            par = dd & 1
            q_d = (dd * v - 4 * par) >> 3
            mbase = par * n_per_w

            def build(c, cc):
                sl = pl.ds(c * _LANES, _LANES)
                idx_v[sl] = mrow_v[pl.ds(mbase + c * _LANES, _LANES)] + q_d
                return cc

            lax.fori_loop(0, n_chunks, build, 0)
            gslot = par * n_per_w
            cp = pltpu.async_copy(
                tab_hbm.at[idx_v], gbuf_v.at[pl.ds(gslot, n_per_w)], sem
            )

            @pl.when(dd > 0)
            def _():
                extract(dd - 1, 1 - par, (1 - par) * n_per_w)

            cp.wait()
            return carry

        lax.fori_loop(0, d, step, 0)
        extract(d - 1, (d - 1) & 1, ((d - 1) & 1) * n_per_w)

        pltpu.sync_copy(wbuf_v, out_hbm.at[pl.ds(wid * rows_per_w, rows_per_w)])

    return k(ids, table_t8)


def kernel(input_ids, table):
    b, s, h = input_ids.shape
    d = table.shape[1]
    tab_t8 = table.T.reshape(-1, 8)
    out = _mhe_lookup(input_ids, tab_t8, b=b, s=s, h=h, d=d)
    one_f = table[0, 0] * 0.0 + 1.0
    return out.reshape(b, s, h, d) * one_f


# speedup vs baseline: 3.2267x; 3.2267x over previous
import functools

import jax
import jax.numpy as jnp
import numpy as np
from jax import lax
from jax.experimental import pallas as pl
from jax.experimental.pallas import tpu as pltpu
from jax.experimental.pallas import tpu_sc as plsc

_VOCAB_SIZES = [100003, 100019, 100043, 100049, 100057, 100069, 100103, 100109]
_OFFSETS = [int(x) for x in np.cumsum([0] + _VOCAB_SIZES[:-1])]

_NUM_CORES = 2
_NUM_SUBCORES = 16
_NUM_WORKERS = _NUM_CORES * _NUM_SUBCORES
_LANES = 16
_CHUNK = 128
_GROUPS = 4
_H = 8


@functools.partial(jax.jit, static_argnames=("b", "s", "h", "d"))
def _mhe_lookup(ids, table, *, b, s, h, d):
    n = b * s * h
    n_per_w = n // _NUM_WORKERS
    s_per_w = n_per_w // h
    rows_per_w = n_per_w // _GROUPS
    chunks_per_group = rows_per_w // _CHUNK
    idx_rows = n_per_w // _CHUNK
    mesh = plsc.VectorSubcoreMesh(core_axis_name="c", subcore_axis_name="s")

    @functools.partial(
        pl.kernel,
        mesh=mesh,
        out_type=jax.ShapeDtypeStruct((n // _GROUPS, _GROUPS * d), jnp.float32),
        scratch_types=[
            pltpu.VMEM((idx_rows, _CHUNK), jnp.int32),
            pltpu.VMEM((idx_rows, _CHUNK), jnp.int32),
            pltpu.VMEM((rows_per_w, d), jnp.float32),
            pltpu.SemaphoreType.DMA,
        ],
        compiler_params=pltpu.CompilerParams(
            use_tc_tiling_on_sc=False, needs_layout_passes=False
        ),
    )
    def k(ids_hbm, table_hbm, out_hbm, idx8_v, idx_v, rows_v, sem):
        del idx8_v
        wid = lax.axis_index("s") * _NUM_CORES + lax.axis_index("c")

        pltpu.sync_copy(ids_hbm.at[pl.ds(wid * idx_rows, idx_rows)], idx_v)

        iota = lax.iota(jnp.int32, _LANES)
        parity = iota & 1
        for j in range(_GROUPS):
            off_j = _OFFSETS[j] + (_OFFSETS[j + _GROUPS] - _OFFSETS[j]) * parity
            for c in range(chunks_per_group):
                row = j * chunks_per_group + c
                for t in range(_CHUNK // _LANES):
                    sl = pl.ds(16 * t, _LANES)
                    idx_v[row, sl] = idx_v[row, sl] + off_j

        out_base = wid * rows_per_w
        for j in range(_GROUPS):
            copies = []
            for c in range(chunks_per_group):
                row = j * chunks_per_group + c
                copies.append(
                    pltpu.async_copy(
                        table_hbm.at[idx_v.at[row]],
                        rows_v.at[pl.ds(c * _CHUNK, _CHUNK)],
                        sem,
                    )
                )
            for cp in copies:
                cp.wait()
            pltpu.sync_copy(
                rows_v,
                out_hbm.at[pl.ds(out_base, rows_per_w), pl.ds(j * d, d)],
            )

    return k(ids, table)


def kernel(input_ids, table):
    b, s, h = input_ids.shape
    d = table.shape[1]
    n = b * s * h
    ids2d = (
        input_ids.reshape(_NUM_WORKERS, n // (_NUM_WORKERS * _GROUPS), _GROUPS)
        .transpose(0, 2, 1)
        .reshape(n // 128, 128)
    )
    one_f = table[0, 0] * 0.0 + 1.0
    out = _mhe_lookup(ids2d, table * one_f, b=b, s=s, h=h, d=d)
    return out.reshape(b, s, h, d) * one_f


# speedup vs baseline: 5.1574x; 1.5983x over previous
import functools

import jax
import jax.numpy as jnp
import numpy as np
from jax import lax
from jax.experimental import pallas as pl
from jax.experimental.pallas import tpu as pltpu
from jax.experimental.pallas import tpu_sc as plsc

_VOCAB_SIZES = [100003, 100019, 100043, 100049, 100057, 100069, 100103, 100109]
_OFFSETS = [int(x) for x in np.cumsum([0] + _VOCAB_SIZES[:-1])]

_NUM_CORES = 2
_NUM_SUBCORES = 16
_NUM_WORKERS = _NUM_CORES * _NUM_SUBCORES
_LANES = 16
_CHUNK = 128
_GROUPS = 4
_H = 8


@functools.partial(jax.jit, static_argnames=("b", "s", "h", "d"))
def _mhe_lookup(ids, table, *, b, s, h, d):
    n = b * s * h
    n_per_w = n // _NUM_WORKERS
    s_per_w = n_per_w // h
    rows_per_w = n_per_w // _GROUPS
    chunks_per_group = rows_per_w // _CHUNK
    idx_rows = n_per_w // _CHUNK
    mesh = plsc.VectorSubcoreMesh(core_axis_name="c", subcore_axis_name="s")

    @functools.partial(
        pl.kernel,
        mesh=mesh,
        out_type=jax.ShapeDtypeStruct((n // _GROUPS, _GROUPS * d), jnp.float32),
        scratch_types=[
            pltpu.VMEM((idx_rows, _CHUNK), jnp.int32),
            pltpu.VMEM((idx_rows, _CHUNK), jnp.int32),
            pltpu.VMEM((rows_per_w, d), jnp.float32),
            pltpu.SemaphoreType.DMA,
        ],
        compiler_params=pltpu.CompilerParams(
            use_tc_tiling_on_sc=False, needs_layout_passes=False
        ),
    )
    def k(ids_hbm, table_hbm, out_hbm, idx8_v, idx_v, rows_v, sem):
        del idx8_v
        wid = lax.axis_index("s") * _NUM_CORES + lax.axis_index("c")

        pltpu.sync_copy(ids_hbm.at[pl.ds(wid * idx_rows, idx_rows)], idx_v)

        iota = lax.iota(jnp.int32, _LANES)
        parity = iota & 1
        for j in range(_GROUPS):
            off_j = _OFFSETS[j] + (_OFFSETS[j + _GROUPS] - _OFFSETS[j]) * parity
            for c in range(chunks_per_group):
                row = j * chunks_per_group + c
                for t in range(_CHUNK // _LANES):
                    sl = pl.ds(16 * t, _LANES)
                    idx_v[row, sl] = idx_v[row, sl] + off_j

        out_base = wid * rows_per_w
        for j in range(_GROUPS):
            copies = []
            for c in range(chunks_per_group):
                row = j * chunks_per_group + c
                copies.append(
                    pltpu.async_copy(
                        table_hbm.at[idx_v.at[row]],
                        rows_v.at[pl.ds(c * _CHUNK, _CHUNK)],
                        sem,
                    )
                )
            for cp in copies:
                cp.wait()
            pltpu.sync_copy(
                rows_v,
                out_hbm.at[pl.ds(out_base, rows_per_w), pl.ds(j * d, d)],
            )

    return k(ids, table)


def kernel(input_ids, table):
    b, s, h = input_ids.shape
    d = table.shape[1]
    n = b * s * h
    ids2d = (
        input_ids.reshape(_NUM_WORKERS, n // (_NUM_WORKERS * _GROUPS), _GROUPS)
        .transpose(0, 2, 1)
        .reshape(n // 128, 128)
    )
    out = _mhe_lookup(ids2d, table, b=b, s=s, h=h, d=d)
    one_f = table[0, 0] * 0.0 + 1.0
    return out.reshape(b, s, h, d) * one_f


# speedup vs baseline: 5.2854x; 1.0248x over previous
import functools

import jax
import jax.numpy as jnp
import numpy as np
from jax import lax
from jax.experimental import pallas as pl
from jax.experimental.pallas import tpu as pltpu
from jax.experimental.pallas import tpu_sc as plsc

_VOCAB_SIZES = [100003, 100019, 100043, 100049, 100057, 100069, 100103, 100109]
_OFFSETS = np.cumsum([0] + _VOCAB_SIZES[:-1]).astype(np.int32)

_NUM_CORES = 2
_NUM_SUBCORES = 16
_NUM_WORKERS = _NUM_CORES * _NUM_SUBCORES
_LANES = 16
_GATHER_CHUNK = 128


@functools.partial(jax.jit, static_argnames=("n", "d"))
def _mhe_lookup(flat_ids, offsets16, table, *, n, d):
    n_per_w = n // _NUM_WORKERS
    n_chunks = n_per_w // _GATHER_CHUNK
    mesh = plsc.VectorSubcoreMesh(core_axis_name="c", subcore_axis_name="s")

    @functools.partial(
        pl.kernel,
        mesh=mesh,
        out_type=jax.ShapeDtypeStruct((n, d), jnp.float32),
        scratch_types=[
            pltpu.VMEM((n_per_w,), jnp.int32),
            pltpu.VMEM((_LANES,), jnp.int32),
            pltpu.VMEM((n_per_w, d), jnp.float32),
            pltpu.SemaphoreType.DMA,
        ],
        compiler_params=pltpu.CompilerParams(use_tc_tiling_on_sc=False),
    )
    def k(ids_hbm, off_hbm, table_hbm, out_hbm, idx_v, off_v, rows_v, sem):
        wid = lax.axis_index("s") * _NUM_CORES + lax.axis_index("c")
        base = wid * n_per_w
        pltpu.sync_copy(ids_hbm.at[pl.ds(base, n_per_w)], idx_v)
        pltpu.sync_copy(off_hbm, off_v)
        off = off_v[...]

        def shift_body(j, carry):
            sl = pl.ds(j * _LANES, _LANES)
            idx_v[sl] = idx_v[sl] + off
            return carry

        lax.fori_loop(0, n_per_w // _LANES, shift_body, 0)

        copies = []
        for c in range(n_chunks):
            sl = pl.ds(c * _GATHER_CHUNK, _GATHER_CHUNK)
            copies.append(
                pltpu.async_copy(table_hbm.at[idx_v.at[sl]], rows_v.at[sl], sem)
            )
        for cp in copies:
            cp.wait()
        pltpu.sync_copy(rows_v, out_hbm.at[pl.ds(base, n_per_w)])

    return k(flat_ids, offsets16, table)


def kernel(input_ids, table):
    b, s, h = input_ids.shape
    d = table.shape[1]
    n = b * s * h
    flat_ids = input_ids.reshape(n)
    offsets16 = jnp.asarray(np.tile(_OFFSETS, _LANES // len(_OFFSETS)))
    out = _mhe_lookup(flat_ids, offsets16, table, n=n, d=d)
    return out.reshape(b, s, h, d)
